# baseline (speedup 1.0000x reference)
import jax
import jax.numpy as jnp
from jax import lax
from jax.experimental import pallas as pl
from jax.experimental.pallas import tpu as pltpu
from jax.experimental.pallas import tpu_sc as plsc

_NUM_BUCKETS = 1000000
_EMBED_DIM = 64
_BATCH = 16384

_info = plsc.get_sparse_core_info()
_NC, _NS, _L = _info.num_cores, _info.num_subcores, _info.num_lanes
_NW = _NC * _NS
_B_PER_W = _BATCH // _NW
_QS = 4
_NB = 3
_NQ = _B_PER_W // _QS
_HALF = _B_PER_W // 2
_IDX_PAD = _B_PER_W + _L


def _embed_body(x_hbm, tableT_hbm, outT_hbm, idx_v, blocks, stage, sem):
    wid = lax.axis_index("s") * _NC + lax.axis_index("c")
    base = wid * _B_PER_W

    pltpu.sync_copy(x_hbm.at[pl.ds(base, _B_PER_W)], idx_v.at[pl.ds(0, _B_PER_W)])
    for i in range(_B_PER_W // _L):
        sl = pl.ds(i * _L, _L)
        idx_v[sl] = lax.rem(idx_v[sl], _NUM_BUCKETS)

    def fire(q, slot):
        chunk = idx_v[pl.ds(q * _QS, _L)]
        for k in range(_QS):
            b = chunk[k]
            col = pl.multiple_of((b >> 7) << 7, 128)
            pltpu.async_copy(
                tableT_hbm.at[:, pl.ds(col, 128)], blocks.at[slot, k], sem
            )

    def wait_quad(slot):
        for k in range(_QS):
            pltpu.make_async_copy(
                tableT_hbm.at[:, pl.ds(0, 128)], blocks.at[slot, k], sem
            ).wait()

    def extract(q, slot):
        chunk = idx_v[pl.ds(q * _QS, _L)]
        for k in range(_QS):
            b = chunk[k]
            lanes = jnp.full((_L,), b & 127, jnp.int32)
            cols = jnp.full((_L,), (q * _QS + k) & (_HALF - 1), jnp.int32)
            for r in range(_EMBED_DIM // _L):
                rows = lax.iota(jnp.int32, _L) + r * _L
                vals = plsc.load_gather(blocks.at[slot, k], [rows, lanes])
                plsc.store_scatter(stage, [rows, cols], vals)

    for s0 in range(_NB):
        fire(s0, s0)

    mid_quad = _HALF // _QS

    def step(q3, carry):
        for m in range(_NB):
            q = q3 * _NB + m
            wait_quad(m)

            @pl.when(q == mid_quad)
            def _():
                pltpu.sync_copy(stage, outT_hbm.at[:, pl.ds(base, _HALF)])

            extract(q, m)

            @pl.when(q + _NB < _NQ)
            def _():
                fire(q + _NB, m)

        return carry

    trips = _NQ // _NB
    lax.fori_loop(0, trips, step, 0)
    for q in range(trips * _NB, _NQ):
        m = q % _NB
        wait_quad(m)
        extract(q, m)

    pltpu.sync_copy(stage, outT_hbm.at[:, pl.ds(base + _HALF, _HALF)])


@jax.jit
def kernel(x, table):
    fn = pl.kernel(
        _embed_body,
        out_type=jax.ShapeDtypeStruct((_EMBED_DIM, _BATCH), jnp.float32),
        mesh=plsc.VectorSubcoreMesh(core_axis_name="c", subcore_axis_name="s"),
        scratch_types=[
            pltpu.VMEM((_IDX_PAD,), jnp.int32),
            pltpu.VMEM((_NB, _QS, _EMBED_DIM, 128), jnp.float32),
            pltpu.VMEM((_EMBED_DIM, _HALF), jnp.float32),
            pltpu.SemaphoreType.DMA,
        ],
        compiler_params=pltpu.CompilerParams(needs_layout_passes=False),
    )
    outT = fn(x, table.T)
    return outT.T
